# baseline (speedup 1.0000x reference)
import functools

import jax
import jax.numpy as jnp
from jax import lax
from jax.experimental import pallas as pl
from jax.experimental.pallas import tpu as pltpu
from jax.experimental.pallas import tpu_sc as plsc

NC = 2
NS = 16
L = 16
NW = NC * NS

U_STEP = 4


def _make_kernel(NP, K, D, n_feat):
    C = NP // NW
    n_steps = C // U_STEP
    E = U_STEP * K
    DV = D // L

    mesh = plsc.VectorSubcoreMesh(core_axis_name="c", subcore_axis_name="s")

    @functools.partial(
        pl.kernel,
        out_type=jax.ShapeDtypeStruct((NP, D), jnp.float32),
        mesh=mesh,
        scratch_types=[
            pltpu.VMEM((n_steps, E), jnp.int32),
            pltpu.VMEM((n_steps, E), jnp.float32),
            pltpu.VMEM((2, E, D), jnp.float32),
            pltpu.VMEM((C, D), jnp.float32),
            pltpu.SemaphoreType.DMA,
            pltpu.SemaphoreType.DMA,
        ],
    )
    def kern(feat_hbm, gidx_hbm, w_hbm, out_hbm,
             idx_v, w_v, rows_v, out_v, sem0, sem1):
        wid = lax.axis_index("s") * NC + lax.axis_index("c")
        base_u = pl.multiple_of(wid * C, 8)
        base_row = pl.multiple_of(wid * n_steps, 8)

        pltpu.sync_copy(gidx_hbm.at[pl.ds(base_row, n_steps), :], idx_v)
        pltpu.sync_copy(w_hbm.at[pl.ds(base_row, n_steps), :], w_v)

        sems = (sem0, sem1)

        def start_gather(s, b):
            pltpu.async_copy(feat_hbm.at[idx_v.at[s]], rows_v.at[b], sems[b])

        def wait_gather(b):
            pltpu.make_async_copy(feat_hbm.at[idx_v.at[0]],
                                  rows_v.at[b], sems[b]).wait()

        def compute(s, b):
            def user(u, c):
                acc = [jnp.zeros((L,), jnp.float32) for _ in range(DV)]
                wv = [w_v[s, pl.ds(u * K + j * L, L)] for j in range(K // L)]
                for k in range(K):
                    e = u * K + k
                    w = wv[k // L][k % L]
                    for d in range(DV):
                        acc[d] = acc[d] + w * rows_v[b, e, pl.ds(d * L, L)]
                row = s * U_STEP + u
                for d in range(DV):
                    out_v[row, pl.ds(d * L, L)] = acc[d]
                return c

            lax.fori_loop(0, U_STEP, user, 0)

        start_gather(0, 0)

        def pair(p, carry):
            s0 = 2 * p
            start_gather(s0 + 1, 1)
            wait_gather(0)
            compute(s0, 0)
            start_gather(jnp.minimum(s0 + 2, n_steps - 1), 0)
            wait_gather(1)
            compute(s0 + 1, 1)
            return carry

        lax.fori_loop(0, n_steps // 2, pair, 0)
        wait_gather(0)

        pltpu.sync_copy(out_v, out_hbm.at[pl.ds(base_u, C), :])

    return kern


def kernel(features, user_graph, user_matrix):
    N, K = user_graph.shape
    n_feat, D = features.shape
    chunk = NW * U_STEP * 8
    NP = ((N + chunk - 1) // chunk) * chunk
    E = U_STEP * K

    gidx = jnp.reshape(user_graph.astype(jnp.int32), (N * K,))
    w = jnp.reshape(user_matrix.astype(jnp.float32), (N * K,))
    pad = NP * K - N * K
    if pad:
        gidx = jnp.pad(gidx, (0, pad))
        w = jnp.pad(w, (0, pad))
    gidx = gidx.reshape(NP * K // E, E)
    w = w.reshape(NP * K // E, E)

    out = _make_kernel(NP, K, D, n_feat)(features.astype(jnp.float32), gidx, w)
    return out[:N]
